# Initial kernel scaffold; baseline (speedup 1.0000x reference)
import jax
import jax.numpy as jnp
from jax.experimental import pallas as pl


def kernel(x, edge_index, W1, att_src1, att_dst1, bias1, W2, att_src2, att_dst2, bias2):
    raise NotImplementedError("write your pallas kernel here")


# speedup vs baseline: 8.0581x; 8.0581x over previous
import functools

import jax
import jax.numpy as jnp
from jax import lax
from jax.experimental import pallas as pl
from jax.experimental.pallas import tpu as pltpu
from jax.experimental.pallas import tpu_sc as plsc

N = 10000
IN = 128
H1, C1 = 8, 128
D1A = C1 + 16
D2 = 64
D2A = 80

NC, NS = 2, 16
NW = NC * NS
CH = 688
NCH = 15
EW = CH * NCH
EP = NW * EW
SR = N // NS
NCH_E = 2 * NCH
NE = 5120
SRE = NE // NS
W80 = 80

_i32 = jnp.int32
_f32 = jnp.float32


def _iota16():
    return jax.lax.iota(_i32, 16)


def _tc_a_body(x_ref, w1_ref, as_ref, ad_ref, *out_refs):
    x = x_ref[...]
    s_parts = []
    d_parts = []
    for h in range(H1):
        w = w1_ref[:, h * C1:(h + 1) * C1]
        ph = jnp.dot(x, w, preferred_element_type=_f32)
        ones = jnp.ones((ph.shape[0], 1), _f32)
        zpad = jnp.zeros((ph.shape[0], 31), _f32)
        out_refs[h][...] = ph[:, :W80]
        out_refs[H1 + h][...] = jnp.concatenate([ph[:, W80:], ones, zpad],
                                                axis=1)
        s_parts.append((ph * as_ref[h][None, :]).sum(axis=1, keepdims=True))
        d_parts.append((ph * ad_ref[h][None, :]).sum(axis=1, keepdims=True))
    out_refs[2 * H1][...] = jnp.concatenate(s_parts + d_parts, axis=1)


def _tc_a(x, W1, att_src1, att_dst1):
    R = 1000
    grid = (N // R,)
    outs = [jax.ShapeDtypeStruct((N, W80), _f32) for _ in range(2 * H1)]
    outs.append(jax.ShapeDtypeStruct((N, 16), _f32))
    out_specs = [pl.BlockSpec((R, W80), lambda i: (i, 0))
                 for _ in range(2 * H1)]
    out_specs.append(pl.BlockSpec((R, 16), lambda i: (i, 0)))
    return pl.pallas_call(
        _tc_a_body,
        grid=grid,
        in_specs=[
            pl.BlockSpec((R, IN), lambda i: (i, 0)),
            pl.BlockSpec((IN, H1 * C1), lambda i: (0, 0)),
            pl.BlockSpec((H1, C1), lambda i: (0, 0)),
            pl.BlockSpec((H1, C1), lambda i: (0, 0)),
        ],
        out_specs=out_specs,
        out_shape=outs,
    )(x, W1, att_src1, att_dst1)


def _sc_b_body(src_ref, dst_ref, asd_ref, et_ref,
               srcc, dstc, asr, adr, ehv):
    c = lax.axis_index("c")
    s = lax.axis_index("s")
    wid = s * NC + c
    gbase = wid * EW
    iota = _iota16()

    def chunk(k, carry):
        off = gbase + k * CH
        pltpu.sync_copy(src_ref.at[pl.ds(off, CH)], srcc)
        pltpu.sync_copy(dst_ref.at[pl.ds(off, CH)], dstc)
        pltpu.sync_copy(asd_ref.at[srcc], asr)
        pltpu.sync_copy(asd_ref.at[dstc], adr)
        for h in range(H1):
            def body(g, carry2, h=h):
                rows = g * 16 + iota
                a_s = plsc.load_gather(asr, [rows, jnp.full((16,), h, _i32)])
                a_d = plsc.load_gather(adr, [rows, jnp.full((16,), 8 + h, _i32)])
                a = a_s + a_d
                a = jnp.where(a >= 0, a, 0.2 * a)
                e = jnp.exp(a)
                gid = off + rows
                e = jnp.where(gid < (320000 + N), e, jnp.zeros((16,), _f32))
                ehv[pl.ds(g * 16, 16)] = e
                return carry2
            lax.fori_loop(0, CH // 16, body, 0)
            pltpu.sync_copy(ehv, et_ref.at[h, pl.ds(off, CH)])
        return carry

    lax.fori_loop(0, NCH, chunk, 0)


def _sc_b(srcP, dstP, asd1):
    mesh = plsc.VectorSubcoreMesh(core_axis_name="c", subcore_axis_name="s")
    kern = pl.kernel(
        _sc_b_body,
        out_type=jax.ShapeDtypeStruct((H1, EP), _f32),
        mesh=mesh,
        compiler_params=pltpu.CompilerParams(use_tc_tiling_on_sc=False, needs_layout_passes=False),
        scratch_types=[
            pltpu.VMEM((CH,), _i32),
            pltpu.VMEM((CH,), _i32),
            pltpu.VMEM((CH, 16), _f32),
            pltpu.VMEM((CH, 16), _f32),
            pltpu.VMEM((CH,), _f32),
        ],
    )
    return kern(srcP, dstP, asd1)


def _scale_rows(rows, ev, width):
    nv = width // 16
    iota = _iota16()

    def body(g, carry):
        evec = ev[pl.ds(g * 16, 16)]
        for lane in range(16):
            e = jnp.full((16,), evec[lane], _f32)
            jv = jnp.full((16,), g * 16 + lane, _i32)
            for cblk in range(nv):
                cols = cblk * 16 + iota
                v = plsc.load_gather(rows, [jv, cols])
                plsc.store_scatter(rows, [jv, cols], v * e)
        return carry

    lax.fori_loop(0, CH // 16, body, 0)


def _sc_c_body(src_ref, dst_ref, et_ref, zeros_ref, *rest):
    t0_refs = rest[:H1]
    t1_refs = rest[H1:2 * H1]
    u0_ref = rest[2 * H1]
    u1_ref = rest[2 * H1 + 1]
    srcc, dstc, ev, rows, ush = rest[2 * H1 + 2:]
    c = lax.axis_index("c")
    s = lax.axis_index("s")
    gbase = s * (NCH_E * CH)
    for h in range(H1):
        if h == 0:
            pltpu.sync_copy(zeros_ref.at[pl.ds(s * SR, SR)],
                            ush.at[pl.ds(s * SR, SR)])
        plsc.subcore_barrier()

        def chunk(k, carry, h=h):
            off = gbase + k * CH
            pltpu.sync_copy(src_ref.at[pl.ds(off, CH)], srcc)
            pltpu.sync_copy(dst_ref.at[pl.ds(off, CH)], dstc)
            pltpu.sync_copy(et_ref.at[h, pl.ds(off, CH)], ev)
            @pl.when(c == 0)
            def _():
                pltpu.sync_copy(t0_refs[h].at[srcc], rows)
            @pl.when(c == 1)
            def _():
                pltpu.sync_copy(t1_refs[h].at[srcc], rows)
            _scale_rows(rows, ev, W80)
            pltpu.sync_copy(rows, ush.at[dstc], add=True)
            return carry

        lax.fori_loop(0, NCH_E, chunk, 0)
        plsc.subcore_barrier()
        @pl.when(c == 0)
        def _(h=h):
            pltpu.sync_copy(ush.at[pl.ds(s * SR, SR)],
                            u0_ref.at[h, pl.ds(s * SR, SR)])
        @pl.when(c == 1)
        def _(h=h):
            pltpu.sync_copy(ush.at[pl.ds(s * SR, SR)],
                            u1_ref.at[h, pl.ds(s * SR, SR)])
        if h < H1 - 1:
            pltpu.sync_copy(zeros_ref.at[pl.ds(s * SR, SR)],
                            ush.at[pl.ds(s * SR, SR)])


def _sc_c(srcP, dstP, eT, tables, zerosC):
    mesh = plsc.VectorSubcoreMesh(core_axis_name="c", subcore_axis_name="s")
    kern = pl.kernel(
        _sc_c_body,
        out_type=[jax.ShapeDtypeStruct((H1, N, W80), _f32),
                  jax.ShapeDtypeStruct((H1, N, W80), _f32)],
        mesh=mesh,
        compiler_params=pltpu.CompilerParams(use_tc_tiling_on_sc=False, needs_layout_passes=False),
        scratch_types=[
            pltpu.VMEM((CH,), _i32),
            pltpu.VMEM((CH,), _i32),
            pltpu.VMEM((CH,), _f32),
            pltpu.VMEM((CH, W80), _f32),
            pltpu.VMEM_SHARED((N, W80), _f32),
        ],
    )
    return kern(srcP, dstP, eT, zerosC, *tables)


def _tc_d_body(ul_ref, ur_ref, w2_ref, as2_ref, ad2_ref, b1_ref,
               h2_ref, asd2_ref):
    parts = []
    for h in range(H1):
        ul = ul_ref[h]
        ur = ur_ref[h]
        den = ur[:, 48:49] + 1e-16
        xh = jnp.concatenate([ul, ur[:, :48]], axis=1) / den + b1_ref[h][None, :]
        parts.append(jnp.maximum(xh, 0.0))
    x1 = jnp.concatenate(parts, axis=1)
    h2 = jnp.dot(x1, w2_ref[...], preferred_element_type=_f32)
    s2 = (h2 * as2_ref[...]).sum(axis=1, keepdims=True)
    d2 = (h2 * ad2_ref[...]).sum(axis=1, keepdims=True)
    ones = jnp.ones((h2.shape[0], 1), _f32)
    zp = jnp.zeros((h2.shape[0], 15), _f32)
    h2_ref[...] = jnp.concatenate([h2, ones, zp], axis=1)
    asd2_ref[...] = jnp.concatenate([s2, d2, jnp.zeros((h2.shape[0], 14), _f32)],
                                    axis=1)


def _tc_d(UL, UR, W2, att_src2, att_dst2, bias1):
    R = 1000
    grid = (N // R,)
    return pl.pallas_call(
        _tc_d_body,
        grid=grid,
        in_specs=[
            pl.BlockSpec((H1, R, W80), lambda i: (0, i, 0)),
            pl.BlockSpec((H1, R, W80), lambda i: (0, i, 0)),
            pl.BlockSpec((H1 * C1, D2), lambda i: (0, 0)),
            pl.BlockSpec((1, D2), lambda i: (0, 0)),
            pl.BlockSpec((1, D2), lambda i: (0, 0)),
            pl.BlockSpec((H1, C1), lambda i: (0, 0)),
        ],
        out_specs=[
            pl.BlockSpec((R, D2A), lambda i: (i, 0)),
            pl.BlockSpec((R, 16), lambda i: (i, 0)),
        ],
        out_shape=[
            jax.ShapeDtypeStruct((N, D2A), _f32),
            jax.ShapeDtypeStruct((N, 16), _f32),
        ],
    )(UL, UR, W2, att_src2, att_dst2, bias1)


def _sc_e_body(src_ref, dst_ref, asd_ref, h2_ref, zeros_ref, u_ref,
               srcc, dstc, asr, adr, ev, rows, ush):
    c = lax.axis_index("c")
    s = lax.axis_index("s")
    gbase = s * (NCH_E * CH)
    iota = _iota16()
    pltpu.sync_copy(zeros_ref.at[pl.ds(s * SRE, SRE)],
                    ush.at[pl.ds(s * SRE, SRE)])
    plsc.subcore_barrier()
    def chunk(k, carry):
        off = gbase + k * CH
        pltpu.sync_copy(src_ref.at[pl.ds(off, CH)], srcc)
        pltpu.sync_copy(dst_ref.at[pl.ds(off, CH)], dstc)
        pltpu.sync_copy(asd_ref.at[srcc], asr)
        pltpu.sync_copy(asd_ref.at[dstc], adr)

        def body(g, carry2):
            rws = g * 16 + iota
            a_s = plsc.load_gather(asr, [rws, jnp.zeros((16,), _i32)])
            a_d = plsc.load_gather(adr, [rws, jnp.ones((16,), _i32)])
            a = a_s + a_d
            a = jnp.where(a >= 0, a, 0.2 * a)
            e = jnp.exp(a)
            gid = off + rws
            e = jnp.where(gid < (320000 + N), e, jnp.zeros((16,), _f32))
            ev[pl.ds(g * 16, 16)] = e
            d = dstc[pl.ds(g * 16, 16)] - c * 5000
            d = jnp.where((d >= 0) & (d < 5000), d,
                          jnp.full((16,), 5000, _i32))
            dstc[pl.ds(g * 16, 16)] = d
            return carry2

        lax.fori_loop(0, CH // 16, body, 0)
        pltpu.sync_copy(h2_ref.at[srcc], rows)
        _scale_rows(rows, ev, D2A)
        pltpu.sync_copy(rows, ush.at[dstc], add=True)
        return carry

    lax.fori_loop(0, NCH_E, chunk, 0)
    plsc.subcore_barrier()
    pltpu.sync_copy(ush.at[pl.ds(s * SRE, SRE)],
                    u_ref.at[c, pl.ds(s * SRE, SRE)])


def _sc_e(srcP, dstP, asd2, h2aug, zerosB):
    mesh = plsc.VectorSubcoreMesh(core_axis_name="c", subcore_axis_name="s")
    kern = pl.kernel(
        _sc_e_body,
        out_type=jax.ShapeDtypeStruct((NC, NE, D2A), _f32),
        mesh=mesh,
        compiler_params=pltpu.CompilerParams(use_tc_tiling_on_sc=False, needs_layout_passes=False),
        scratch_types=[
            pltpu.VMEM((CH,), _i32),
            pltpu.VMEM((CH,), _i32),
            pltpu.VMEM((CH, 16), _f32),
            pltpu.VMEM((CH, 16), _f32),
            pltpu.VMEM((CH,), _f32),
            pltpu.VMEM((CH, D2A), _f32),
            pltpu.VMEM_SHARED((NE, D2A), _f32),
        ],
    )
    return kern(srcP, dstP, asd2, h2aug, zerosB)


def _tc_f_body(u0_ref, b2_ref, out_ref):
    u = u0_ref[...]
    den = u[:, D2:D2 + 1] + 1e-16
    out_ref[...] = u[:, :D2] / den + b2_ref[...]


def _tc_f(V0, bias2):
    R = 1000
    return pl.pallas_call(
        _tc_f_body,
        grid=(N // R,),
        in_specs=[
            pl.BlockSpec((R, D2A), lambda i: (i, 0)),
            pl.BlockSpec((1, D2), lambda i: (0, 0)),
        ],
        out_specs=pl.BlockSpec((R, D2), lambda i: (i, 0)),
        out_shape=jax.ShapeDtypeStruct((N, D2), _f32),
    )(V0, bias2)


@jax.jit
def _run(x, edge_index, W1, att_src1, att_dst1, bias1,
         W2, att_src2, att_dst2, bias2):
    loop = jnp.arange(N, dtype=edge_index.dtype)
    pad = jnp.zeros((EP - 320000 - N,), dtype=edge_index.dtype)
    srcP = jnp.concatenate([edge_index[0], loop, pad])
    dstP = jnp.concatenate([edge_index[1], loop, pad])

    outs_a = _tc_a(x, W1, att_src1, att_dst1)
    tables, asd1 = outs_a[:2 * H1], outs_a[2 * H1]

    eT = _sc_b(srcP, dstP, asd1)

    zerosC = jnp.zeros((N, W80), _f32)
    UL, UR = _sc_c(srcP, dstP, eT, tables, zerosC)

    h2aug, asd2 = _tc_d(UL, UR, W2, att_src2, att_dst2,
                        bias1.reshape(H1, C1))

    V = _sc_e(srcP, dstP, asd2, h2aug, zerosC)
    Vcat = jnp.concatenate([V[0, :5000], V[1, :5000]], axis=0)

    out2 = _tc_f(Vcat, bias2.reshape(1, D2))
    return out2[:, :D2 // 2], out2[:, D2 // 2:]


def kernel(x, edge_index, W1, att_src1, att_dst1, bias1,
           W2, att_src2, att_dst2, bias2):
    return _run(x, edge_index, W1, att_src1, att_dst1, bias1,
                W2, att_src2, att_dst2, bias2)


# speedup vs baseline: 17.6405x; 2.1892x over previous
import functools

import jax
import jax.numpy as jnp
from jax import lax
from jax.experimental import pallas as pl
from jax.experimental.pallas import tpu as pltpu
from jax.experimental.pallas import tpu_sc as plsc

N = 10000
IN = 128
H1, C1 = 8, 128
D1A = C1 + 16
D2 = 64
D2A = 80

NC, NS = 2, 16
NW = NC * NS
CH = 688
NCH = 15
EW = CH * NCH
EP = NW * EW
SR = N // NS
NCH_E = 2 * NCH
NE = 5120
SRE = NE // NS
W80 = 80

_i32 = jnp.int32
_f32 = jnp.float32


def _iota16():
    return jax.lax.iota(_i32, 16)


def _tc_a_body(x_ref, w1_ref, as_ref, ad_ref, *out_refs):
    x = x_ref[...]
    s_parts = []
    d_parts = []
    for h in range(H1):
        w = w1_ref[:, h * C1:(h + 1) * C1]
        ph = jnp.dot(x, w, preferred_element_type=_f32)
        ones = jnp.ones((ph.shape[0], 1), _f32)
        zpad = jnp.zeros((ph.shape[0], 31), _f32)
        out_refs[h][...] = ph[:, :W80]
        out_refs[H1 + h][...] = jnp.concatenate([ph[:, W80:], ones, zpad],
                                                axis=1)
        s_parts.append((ph * as_ref[h][None, :]).sum(axis=1, keepdims=True))
        d_parts.append((ph * ad_ref[h][None, :]).sum(axis=1, keepdims=True))
    out_refs[2 * H1][...] = jnp.concatenate(s_parts + d_parts, axis=1)


def _tc_a(x, W1, att_src1, att_dst1):
    R = 1000
    grid = (N // R,)
    outs = [jax.ShapeDtypeStruct((N, W80), _f32) for _ in range(2 * H1)]
    outs.append(jax.ShapeDtypeStruct((N, 16), _f32))
    out_specs = [pl.BlockSpec((R, W80), lambda i: (i, 0))
                 for _ in range(2 * H1)]
    out_specs.append(pl.BlockSpec((R, 16), lambda i: (i, 0)))
    return pl.pallas_call(
        _tc_a_body,
        grid=grid,
        in_specs=[
            pl.BlockSpec((R, IN), lambda i: (i, 0)),
            pl.BlockSpec((IN, H1 * C1), lambda i: (0, 0)),
            pl.BlockSpec((H1, C1), lambda i: (0, 0)),
            pl.BlockSpec((H1, C1), lambda i: (0, 0)),
        ],
        out_specs=out_specs,
        out_shape=outs,
    )(x, W1, att_src1, att_dst1)


def _sc_b_body(src_ref, dst_ref, asd_ref, et_ref,
               srcc, dstc, asr, adr, ehv):
    c = lax.axis_index("c")
    s = lax.axis_index("s")
    wid = s * NC + c
    gbase = wid * EW
    iota = _iota16()

    def chunk(k, carry):
        off = gbase + k * CH
        pltpu.sync_copy(src_ref.at[pl.ds(off, CH)], srcc)
        pltpu.sync_copy(dst_ref.at[pl.ds(off, CH)], dstc)
        pltpu.sync_copy(asd_ref.at[srcc], asr)
        pltpu.sync_copy(asd_ref.at[dstc], adr)
        for h in range(H1):
            def body(g, carry2, h=h):
                rows = g * 16 + iota
                a_s = plsc.load_gather(asr, [rows, jnp.full((16,), h, _i32)])
                a_d = plsc.load_gather(adr, [rows, jnp.full((16,), 8 + h, _i32)])
                a = a_s + a_d
                a = jnp.where(a >= 0, a, 0.2 * a)
                e = jnp.exp(a)
                gid = off + rows
                e = jnp.where(gid < (320000 + N), e, jnp.zeros((16,), _f32))
                ehv[pl.ds(g * 16, 16)] = e
                return carry2
            lax.fori_loop(0, CH // 16, body, 0)
            pltpu.sync_copy(ehv, et_ref.at[h, pl.ds(off, CH)])
        return carry

    lax.fori_loop(0, NCH, chunk, 0)


def _sc_b(srcP, dstP, asd1):
    mesh = plsc.VectorSubcoreMesh(core_axis_name="c", subcore_axis_name="s")
    kern = pl.kernel(
        _sc_b_body,
        out_type=jax.ShapeDtypeStruct((H1, EP), _f32),
        mesh=mesh,
        compiler_params=pltpu.CompilerParams(use_tc_tiling_on_sc=False, needs_layout_passes=False),
        scratch_types=[
            pltpu.VMEM((CH,), _i32),
            pltpu.VMEM((CH,), _i32),
            pltpu.VMEM((CH, 16), _f32),
            pltpu.VMEM((CH, 16), _f32),
            pltpu.VMEM((CH,), _f32),
        ],
    )
    return kern(srcP, dstP, asd1)


def _scale_rows(rows, ev, width):
    nv = width // 16
    iota = _iota16()

    def body(g, carry):
        evec = ev[pl.ds(g * 16, 16)]
        for lane in range(16):
            e = jnp.full((16,), evec[lane], _f32)
            j = g * 16 + lane
            for cblk in range(nv):
                rows[j, pl.ds(cblk * 16, 16)] = rows[j, pl.ds(cblk * 16, 16)] * e
        return carry

    lax.fori_loop(0, CH // 16, body, 0)


def _sc_c_body(src_ref, dst_ref, et_ref, zeros_ref, *rest):
    t0_refs = rest[:H1]
    t1_refs = rest[H1:2 * H1]
    u0_ref = rest[2 * H1]
    u1_ref = rest[2 * H1 + 1]
    srcc, dstc, ev, rows, ush = rest[2 * H1 + 2:]
    c = lax.axis_index("c")
    s = lax.axis_index("s")
    gbase = s * (NCH_E * CH)
    for h in range(H1):
        if h == 0:
            pltpu.sync_copy(zeros_ref.at[pl.ds(s * SR, SR)],
                            ush.at[pl.ds(s * SR, SR)])
        plsc.subcore_barrier()

        def chunk(k, carry, h=h):
            off = gbase + k * CH
            pltpu.sync_copy(src_ref.at[pl.ds(off, CH)], srcc)
            pltpu.sync_copy(dst_ref.at[pl.ds(off, CH)], dstc)
            pltpu.sync_copy(et_ref.at[h, pl.ds(off, CH)], ev)
            @pl.when(c == 0)
            def _():
                pltpu.sync_copy(t0_refs[h].at[srcc], rows)
            @pl.when(c == 1)
            def _():
                pltpu.sync_copy(t1_refs[h].at[srcc], rows)
            _scale_rows(rows, ev, W80)
            pltpu.sync_copy(rows, ush.at[dstc], add=True)
            return carry

        lax.fori_loop(0, NCH_E, chunk, 0)
        plsc.subcore_barrier()
        @pl.when(c == 0)
        def _(h=h):
            pltpu.sync_copy(ush.at[pl.ds(s * SR, SR)],
                            u0_ref.at[h, pl.ds(s * SR, SR)])
        @pl.when(c == 1)
        def _(h=h):
            pltpu.sync_copy(ush.at[pl.ds(s * SR, SR)],
                            u1_ref.at[h, pl.ds(s * SR, SR)])
        if h < H1 - 1:
            pltpu.sync_copy(zeros_ref.at[pl.ds(s * SR, SR)],
                            ush.at[pl.ds(s * SR, SR)])


def _sc_c(srcP, dstP, eT, tables, zerosC):
    mesh = plsc.VectorSubcoreMesh(core_axis_name="c", subcore_axis_name="s")
    kern = pl.kernel(
        _sc_c_body,
        out_type=[jax.ShapeDtypeStruct((H1, N, W80), _f32),
                  jax.ShapeDtypeStruct((H1, N, W80), _f32)],
        mesh=mesh,
        compiler_params=pltpu.CompilerParams(use_tc_tiling_on_sc=False, needs_layout_passes=False),
        scratch_types=[
            pltpu.VMEM((CH,), _i32),
            pltpu.VMEM((CH,), _i32),
            pltpu.VMEM((CH,), _f32),
            pltpu.VMEM((CH, W80), _f32),
            pltpu.VMEM_SHARED((N, W80), _f32),
        ],
    )
    return kern(srcP, dstP, eT, zerosC, *tables)


def _tc_d_body(ul_ref, ur_ref, w2_ref, as2_ref, ad2_ref, b1_ref,
               h2_ref, asd2_ref):
    parts = []
    for h in range(H1):
        ul = ul_ref[h]
        ur = ur_ref[h]
        den = ur[:, 48:49] + 1e-16
        xh = jnp.concatenate([ul, ur[:, :48]], axis=1) / den + b1_ref[h][None, :]
        parts.append(jnp.maximum(xh, 0.0))
    x1 = jnp.concatenate(parts, axis=1)
    h2 = jnp.dot(x1, w2_ref[...], preferred_element_type=_f32)
    s2 = (h2 * as2_ref[...]).sum(axis=1, keepdims=True)
    d2 = (h2 * ad2_ref[...]).sum(axis=1, keepdims=True)
    ones = jnp.ones((h2.shape[0], 1), _f32)
    zp = jnp.zeros((h2.shape[0], 15), _f32)
    h2_ref[...] = jnp.concatenate([h2, ones, zp], axis=1)
    asd2_ref[...] = jnp.concatenate([s2, d2, jnp.zeros((h2.shape[0], 14), _f32)],
                                    axis=1)


def _tc_d(UL, UR, W2, att_src2, att_dst2, bias1):
    R = 1000
    grid = (N // R,)
    return pl.pallas_call(
        _tc_d_body,
        grid=grid,
        in_specs=[
            pl.BlockSpec((H1, R, W80), lambda i: (0, i, 0)),
            pl.BlockSpec((H1, R, W80), lambda i: (0, i, 0)),
            pl.BlockSpec((H1 * C1, D2), lambda i: (0, 0)),
            pl.BlockSpec((1, D2), lambda i: (0, 0)),
            pl.BlockSpec((1, D2), lambda i: (0, 0)),
            pl.BlockSpec((H1, C1), lambda i: (0, 0)),
        ],
        out_specs=[
            pl.BlockSpec((R, D2A), lambda i: (i, 0)),
            pl.BlockSpec((R, 16), lambda i: (i, 0)),
        ],
        out_shape=[
            jax.ShapeDtypeStruct((N, D2A), _f32),
            jax.ShapeDtypeStruct((N, 16), _f32),
        ],
    )(UL, UR, W2, att_src2, att_dst2, bias1)


def _sc_e_body(src_ref, dst_ref, asd_ref, h2_ref, zeros_ref, u_ref,
               srcc, dstc, asr, adr, ev, rows, ush):
    c = lax.axis_index("c")
    s = lax.axis_index("s")
    gbase = s * (NCH_E * CH)
    iota = _iota16()
    pltpu.sync_copy(zeros_ref.at[pl.ds(s * SRE, SRE)],
                    ush.at[pl.ds(s * SRE, SRE)])
    plsc.subcore_barrier()
    def chunk(k, carry):
        off = gbase + k * CH
        pltpu.sync_copy(src_ref.at[pl.ds(off, CH)], srcc)
        pltpu.sync_copy(dst_ref.at[pl.ds(off, CH)], dstc)
        pltpu.sync_copy(asd_ref.at[srcc], asr)
        pltpu.sync_copy(asd_ref.at[dstc], adr)

        def body(g, carry2):
            rws = g * 16 + iota
            a_s = plsc.load_gather(asr, [rws, jnp.zeros((16,), _i32)])
            a_d = plsc.load_gather(adr, [rws, jnp.ones((16,), _i32)])
            a = a_s + a_d
            a = jnp.where(a >= 0, a, 0.2 * a)
            e = jnp.exp(a)
            gid = off + rws
            e = jnp.where(gid < (320000 + N), e, jnp.zeros((16,), _f32))
            ev[pl.ds(g * 16, 16)] = e
            d = dstc[pl.ds(g * 16, 16)] - c * 5000
            d = jnp.where((d >= 0) & (d < 5000), d,
                          jnp.full((16,), 5000, _i32))
            dstc[pl.ds(g * 16, 16)] = d
            return carry2

        lax.fori_loop(0, CH // 16, body, 0)
        pltpu.sync_copy(h2_ref.at[srcc], rows)
        _scale_rows(rows, ev, D2A)
        pltpu.sync_copy(rows, ush.at[dstc], add=True)
        return carry

    lax.fori_loop(0, NCH_E, chunk, 0)
    plsc.subcore_barrier()
    pltpu.sync_copy(ush.at[pl.ds(s * SRE, SRE)],
                    u_ref.at[c, pl.ds(s * SRE, SRE)])


def _sc_e(srcP, dstP, asd2, h2aug, zerosB):
    mesh = plsc.VectorSubcoreMesh(core_axis_name="c", subcore_axis_name="s")
    kern = pl.kernel(
        _sc_e_body,
        out_type=jax.ShapeDtypeStruct((NC, NE, D2A), _f32),
        mesh=mesh,
        compiler_params=pltpu.CompilerParams(use_tc_tiling_on_sc=False, needs_layout_passes=False),
        scratch_types=[
            pltpu.VMEM((CH,), _i32),
            pltpu.VMEM((CH,), _i32),
            pltpu.VMEM((CH, 16), _f32),
            pltpu.VMEM((CH, 16), _f32),
            pltpu.VMEM((CH,), _f32),
            pltpu.VMEM((CH, D2A), _f32),
            pltpu.VMEM_SHARED((NE, D2A), _f32),
        ],
    )
    return kern(srcP, dstP, asd2, h2aug, zerosB)


def _tc_f_body(u0_ref, b2_ref, out_ref):
    u = u0_ref[...]
    den = u[:, D2:D2 + 1] + 1e-16
    out_ref[...] = u[:, :D2] / den + b2_ref[...]


def _tc_f(V0, bias2):
    R = 1000
    return pl.pallas_call(
        _tc_f_body,
        grid=(N // R,),
        in_specs=[
            pl.BlockSpec((R, D2A), lambda i: (i, 0)),
            pl.BlockSpec((1, D2), lambda i: (0, 0)),
        ],
        out_specs=pl.BlockSpec((R, D2), lambda i: (i, 0)),
        out_shape=jax.ShapeDtypeStruct((N, D2), _f32),
    )(V0, bias2)


@jax.jit
def _run(x, edge_index, W1, att_src1, att_dst1, bias1,
         W2, att_src2, att_dst2, bias2):
    loop = jnp.arange(N, dtype=edge_index.dtype)
    pad = jnp.zeros((EP - 320000 - N,), dtype=edge_index.dtype)
    srcP = jnp.concatenate([edge_index[0], loop, pad])
    dstP = jnp.concatenate([edge_index[1], loop, pad])

    outs_a = _tc_a(x, W1, att_src1, att_dst1)
    tables, asd1 = outs_a[:2 * H1], outs_a[2 * H1]

    eT = _sc_b(srcP, dstP, asd1)

    zerosC = jnp.zeros((N, W80), _f32)
    UL, UR = _sc_c(srcP, dstP, eT, tables, zerosC)

    h2aug, asd2 = _tc_d(UL, UR, W2, att_src2, att_dst2,
                        bias1.reshape(H1, C1))

    V = _sc_e(srcP, dstP, asd2, h2aug, zerosC)
    Vcat = jnp.concatenate([V[0, :5000], V[1, :5000]], axis=0)

    out2 = _tc_f(Vcat, bias2.reshape(1, D2))
    return out2[:, :D2 // 2], out2[:, D2 // 2:]


def kernel(x, edge_index, W1, att_src1, att_dst1, bias1,
           W2, att_src2, att_dst2, bias2):
    return _run(x, edge_index, W1, att_src1, att_dst1, bias1,
                W2, att_src2, att_dst2, bias2)


# speedup vs baseline: 20.4826x; 1.1611x over previous
import functools

import jax
import jax.numpy as jnp
from jax import lax
from jax.experimental import pallas as pl
from jax.experimental.pallas import tpu as pltpu
from jax.experimental.pallas import tpu_sc as plsc

N = 10000
IN = 128
H1, C1 = 8, 128
D1A = C1 + 16
D2 = 64
D2A = 80

NC, NS = 2, 16
NW = NC * NS
CH = 688
NCH = 15
EW = CH * NCH
EP = NW * EW
SR = N // NS
NCH_E = 2 * NCH
NE = 5120
SRE = NE // NS
W80 = 80
CHC = 480
NCHC = 20640 // CHC

_i32 = jnp.int32
_f32 = jnp.float32


def _iota16():
    return jax.lax.iota(_i32, 16)


def _tc_a_body(x_ref, w1_ref, as_ref, ad_ref, *out_refs):
    x = x_ref[...]
    s_parts = []
    d_parts = []
    l_parts = []
    r_parts = []
    for h in range(H1):
        w = w1_ref[:, h * C1:(h + 1) * C1]
        ph = jnp.dot(x, w, preferred_element_type=_f32)
        ones = jnp.ones((ph.shape[0], 1), _f32)
        zpad = jnp.zeros((ph.shape[0], 31), _f32)
        l_parts.append(ph[:, :W80][None])
        r_parts.append(jnp.concatenate([ph[:, W80:], ones, zpad], axis=1)[None])
        s_parts.append((ph * as_ref[h][None, :]).sum(axis=1, keepdims=True))
        d_parts.append((ph * ad_ref[h][None, :]).sum(axis=1, keepdims=True))
    out_refs[0][...] = jnp.concatenate(l_parts, axis=0)
    out_refs[1][...] = jnp.concatenate(r_parts, axis=0)
    out_refs[2][...] = jnp.concatenate(s_parts + d_parts, axis=1)


def _tc_a(x, W1, att_src1, att_dst1):
    R = 1000
    grid = (N // R,)
    outs = [jax.ShapeDtypeStruct((H1, N, W80), _f32) for _ in range(2)]
    outs.append(jax.ShapeDtypeStruct((N, 16), _f32))
    out_specs = [pl.BlockSpec((H1, R, W80), lambda i: (0, i, 0))
                 for _ in range(2)]
    out_specs.append(pl.BlockSpec((R, 16), lambda i: (i, 0)))
    return pl.pallas_call(
        _tc_a_body,
        grid=grid,
        in_specs=[
            pl.BlockSpec((R, IN), lambda i: (i, 0)),
            pl.BlockSpec((IN, H1 * C1), lambda i: (0, 0)),
            pl.BlockSpec((H1, C1), lambda i: (0, 0)),
            pl.BlockSpec((H1, C1), lambda i: (0, 0)),
        ],
        out_specs=out_specs,
        out_shape=outs,
    )(x, W1, att_src1, att_dst1)


def _sc_b_body(src_ref, dst_ref, asd_ref, et_ref,
               srcc, dstc, asr, adr, ehv):
    c = lax.axis_index("c")
    s = lax.axis_index("s")
    wid = s * NC + c
    gbase = wid * EW
    iota = _iota16()

    def chunk(k, carry):
        off = gbase + k * CH
        pltpu.sync_copy(src_ref.at[pl.ds(off, CH)], srcc)
        pltpu.sync_copy(dst_ref.at[pl.ds(off, CH)], dstc)
        pltpu.sync_copy(asd_ref.at[srcc], asr)
        pltpu.sync_copy(asd_ref.at[dstc], adr)
        for h in range(H1):
            def body(g, carry2, h=h):
                rows = g * 16 + iota
                a_s = plsc.load_gather(asr, [rows, jnp.full((16,), h, _i32)])
                a_d = plsc.load_gather(adr, [rows, jnp.full((16,), 8 + h, _i32)])
                a = a_s + a_d
                a = jnp.where(a >= 0, a, 0.2 * a)
                e = jnp.exp(a)
                gid = off + rows
                e = jnp.where(gid < (320000 + N), e, jnp.zeros((16,), _f32))
                ehv[pl.ds(g * 16, 16)] = e
                return carry2
            lax.fori_loop(0, CH // 16, body, 0)
            pltpu.sync_copy(ehv, et_ref.at[h, pl.ds(off, CH)])
        return carry

    lax.fori_loop(0, NCH, chunk, 0)


def _sc_b(srcP, dstP, asd1):
    mesh = plsc.VectorSubcoreMesh(core_axis_name="c", subcore_axis_name="s")
    kern = pl.kernel(
        _sc_b_body,
        out_type=jax.ShapeDtypeStruct((H1, EP), _f32),
        mesh=mesh,
        compiler_params=pltpu.CompilerParams(use_tc_tiling_on_sc=False, needs_layout_passes=False),
        scratch_types=[
            pltpu.VMEM((CH,), _i32),
            pltpu.VMEM((CH,), _i32),
            pltpu.VMEM((CH, 16), _f32),
            pltpu.VMEM((CH, 16), _f32),
            pltpu.VMEM((CH,), _f32),
        ],
    )
    return kern(srcP, dstP, asd1)


def _scale_rows(rows, ev, width, count=CH):
    nv = width // 16
    iota = _iota16()

    def body(g, carry):
        evec = ev[pl.ds(g * 16, 16)]
        for lane in range(16):
            e = jnp.full((16,), evec[lane], _f32)
            j = g * 16 + lane
            for cblk in range(nv):
                rows[j, pl.ds(cblk * 16, 16)] = rows[j, pl.ds(cblk * 16, 16)] * e
        return carry

    lax.fori_loop(0, count // 16, body, 0)


def _drain(dummy_hbm, vbuf, sem):
    pltpu.make_async_copy(dummy_hbm, vbuf, sem).wait()


def _sc_c_body(src_ref, dst_ref, et_ref, zeros_ref, dummy_ref, tl_ref, tr_ref,
               u0_ref, u1_ref,
               srcc0, srcc1, dstc0, dstc1, ev0, ev1, rows0, rows1, ush,
               gsem0, gsem1):
    c = lax.axis_index("c")
    s = lax.axis_index("s")
    gbase = s * (NCHC * CHC)
    iota = _iota16()
    srcc = (srcc0, srcc1)
    dstc = (dstc0, dstc1)
    ev = (ev0, ev1)
    rows = (rows0, rows1)
    gsem = (gsem0, gsem1)
    dummy = dummy_ref
    TOT = H1 * NCHC

    def load_idx(t, b):
        h = t // NCHC
        off = gbase + (t % NCHC) * CHC
        pltpu.sync_copy(src_ref.at[pl.ds(off, CHC)], srcc[b])
        pltpu.sync_copy(dst_ref.at[pl.ds(off, CHC)], dstc[b])
        pltpu.sync_copy(et_ref.at[pl.ds(h * EP + off, CHC)], ev[b])
        hbase = h * N

        def shift(g, carry):
            srcc[b][pl.ds(g * 16, 16)] = srcc[b][pl.ds(g * 16, 16)] + hbase
            return carry

        lax.fori_loop(0, CHC // 16, shift, 0)

    def start_gather(b):
        @pl.when(c == 0)
        def _():
            pltpu.async_copy(tl_ref.at[srcc[b]], rows[b], gsem[b])
        @pl.when(c == 1)
        def _():
            pltpu.async_copy(tr_ref.at[srcc[b]], rows[b], gsem[b])

    pltpu.sync_copy(zeros_ref.at[pl.ds(s * SR, SR)],
                    ush.at[pl.ds(s * SR, SR)])
    plsc.subcore_barrier()
    load_idx(0, 0)
    start_gather(0)

    def step(t, carry):
        for b in range(2):
            @pl.when(t % 2 == b)
            def _(b=b):
                nb = 1 - b
                _drain(dummy, rows[b], gsem[b])
                @pl.when(t + 1 < TOT)
                def _():
                    load_idx(t + 1, nb)
                    start_gather(nb)
                _scale_rows(rows[b], ev[b], W80, CHC)
                pltpu.sync_copy(rows[b], ush.at[dstc[b]], add=True)
        @pl.when(t % NCHC == NCHC - 1)
        def _():
            h = t // NCHC
            plsc.subcore_barrier()
            @pl.when(c == 0)
            def _():
                pltpu.sync_copy(ush.at[pl.ds(s * SR, SR)],
                                u0_ref.at[pl.ds(h * N + s * SR, SR)])
            @pl.when(c == 1)
            def _():
                pltpu.sync_copy(ush.at[pl.ds(s * SR, SR)],
                                u1_ref.at[pl.ds(h * N + s * SR, SR)])
            pltpu.sync_copy(zeros_ref.at[pl.ds(s * SR, SR)],
                            ush.at[pl.ds(s * SR, SR)])
            plsc.subcore_barrier()
        return carry

    lax.fori_loop(0, TOT, step, 0)


def _sc_c(srcP, dstP, eT, TL, TR, zerosC, dummyC):
    mesh = plsc.VectorSubcoreMesh(core_axis_name="c", subcore_axis_name="s")
    kern = pl.kernel(
        _sc_c_body,
        out_type=[jax.ShapeDtypeStruct((H1 * N, W80), _f32),
                  jax.ShapeDtypeStruct((H1 * N, W80), _f32)],
        mesh=mesh,
        compiler_params=pltpu.CompilerParams(use_tc_tiling_on_sc=False, needs_layout_passes=False),
        scratch_types=[
            pltpu.VMEM((CHC,), _i32),
            pltpu.VMEM((CHC,), _i32),
            pltpu.VMEM((CHC,), _i32),
            pltpu.VMEM((CHC,), _i32),
            pltpu.VMEM((CHC,), _f32),
            pltpu.VMEM((CHC,), _f32),
            pltpu.VMEM((CHC, W80), _f32),
            pltpu.VMEM((CHC, W80), _f32),
            pltpu.VMEM_SHARED((N, W80), _f32),
            pltpu.SemaphoreType.DMA,
            pltpu.SemaphoreType.DMA,
        ],
    )
    return kern(srcP, dstP, eT, zerosC, dummyC, TL, TR)


def _tc_d_body(ul_ref, ur_ref, w2_ref, as2_ref, ad2_ref, b1_ref,
               h2_ref, asd2_ref):
    parts = []
    for h in range(H1):
        ul = ul_ref[h]
        ur = ur_ref[h]
        den = ur[:, 48:49] + 1e-16
        xh = jnp.concatenate([ul, ur[:, :48]], axis=1) / den + b1_ref[h][None, :]
        parts.append(jnp.maximum(xh, 0.0))
    x1 = jnp.concatenate(parts, axis=1)
    h2 = jnp.dot(x1, w2_ref[...], preferred_element_type=_f32)
    s2 = (h2 * as2_ref[...]).sum(axis=1, keepdims=True)
    d2 = (h2 * ad2_ref[...]).sum(axis=1, keepdims=True)
    ones = jnp.ones((h2.shape[0], 1), _f32)
    zp = jnp.zeros((h2.shape[0], 15), _f32)
    h2_ref[...] = jnp.concatenate([h2, ones, zp], axis=1)
    asd2_ref[...] = jnp.concatenate([s2, d2, jnp.zeros((h2.shape[0], 14), _f32)],
                                    axis=1)


def _tc_d(UL, UR, W2, att_src2, att_dst2, bias1):
    R = 1000
    grid = (N // R,)
    return pl.pallas_call(
        _tc_d_body,
        grid=grid,
        in_specs=[
            pl.BlockSpec((H1, R, W80), lambda i: (0, i, 0)),
            pl.BlockSpec((H1, R, W80), lambda i: (0, i, 0)),
            pl.BlockSpec((H1 * C1, D2), lambda i: (0, 0)),
            pl.BlockSpec((1, D2), lambda i: (0, 0)),
            pl.BlockSpec((1, D2), lambda i: (0, 0)),
            pl.BlockSpec((H1, C1), lambda i: (0, 0)),
        ],
        out_specs=[
            pl.BlockSpec((R, D2A), lambda i: (i, 0)),
            pl.BlockSpec((R, 16), lambda i: (i, 0)),
        ],
        out_shape=[
            jax.ShapeDtypeStruct((N, D2A), _f32),
            jax.ShapeDtypeStruct((N, 16), _f32),
        ],
    )(UL, UR, W2, att_src2, att_dst2, bias1)


def _sc_e_body(src_ref, dst_ref, asd_ref, h2_ref, zeros_ref, u_ref,
               srcc, dstc, asr, adr, ev, rows, ush):
    c = lax.axis_index("c")
    s = lax.axis_index("s")
    gbase = s * (NCH_E * CH)
    iota = _iota16()
    pltpu.sync_copy(zeros_ref.at[pl.ds(s * SRE, SRE)],
                    ush.at[pl.ds(s * SRE, SRE)])
    plsc.subcore_barrier()
    def chunk(k, carry):
        off = gbase + k * CH
        pltpu.sync_copy(src_ref.at[pl.ds(off, CH)], srcc)
        pltpu.sync_copy(dst_ref.at[pl.ds(off, CH)], dstc)
        pltpu.sync_copy(asd_ref.at[srcc], asr)
        pltpu.sync_copy(asd_ref.at[dstc], adr)

        def body(g, carry2):
            rws = g * 16 + iota
            a_s = plsc.load_gather(asr, [rws, jnp.zeros((16,), _i32)])
            a_d = plsc.load_gather(adr, [rws, jnp.ones((16,), _i32)])
            a = a_s + a_d
            a = jnp.where(a >= 0, a, 0.2 * a)
            e = jnp.exp(a)
            gid = off + rws
            e = jnp.where(gid < (320000 + N), e, jnp.zeros((16,), _f32))
            ev[pl.ds(g * 16, 16)] = e
            d = dstc[pl.ds(g * 16, 16)] - c * 5000
            d = jnp.where((d >= 0) & (d < 5000), d,
                          jnp.full((16,), 5000, _i32))
            dstc[pl.ds(g * 16, 16)] = d
            return carry2

        lax.fori_loop(0, CH // 16, body, 0)
        pltpu.sync_copy(h2_ref.at[srcc], rows)
        _scale_rows(rows, ev, D2A)
        pltpu.sync_copy(rows, ush.at[dstc], add=True)
        return carry

    lax.fori_loop(0, NCH_E, chunk, 0)
    plsc.subcore_barrier()
    pltpu.sync_copy(ush.at[pl.ds(s * SRE, SRE)],
                    u_ref.at[c, pl.ds(s * SRE, SRE)])


def _sc_e(srcP, dstP, asd2, h2aug, zerosB):
    mesh = plsc.VectorSubcoreMesh(core_axis_name="c", subcore_axis_name="s")
    kern = pl.kernel(
        _sc_e_body,
        out_type=jax.ShapeDtypeStruct((NC, NE, D2A), _f32),
        mesh=mesh,
        compiler_params=pltpu.CompilerParams(use_tc_tiling_on_sc=False, needs_layout_passes=False),
        scratch_types=[
            pltpu.VMEM((CH,), _i32),
            pltpu.VMEM((CH,), _i32),
            pltpu.VMEM((CH, 16), _f32),
            pltpu.VMEM((CH, 16), _f32),
            pltpu.VMEM((CH,), _f32),
            pltpu.VMEM((CH, D2A), _f32),
            pltpu.VMEM_SHARED((NE, D2A), _f32),
        ],
    )
    return kern(srcP, dstP, asd2, h2aug, zerosB)


def _tc_f_body(u0_ref, b2_ref, out_ref):
    u = u0_ref[...]
    den = u[:, D2:D2 + 1] + 1e-16
    out_ref[...] = u[:, :D2] / den + b2_ref[...]


def _tc_f(V0, bias2):
    R = 1000
    return pl.pallas_call(
        _tc_f_body,
        grid=(N // R,),
        in_specs=[
            pl.BlockSpec((R, D2A), lambda i: (i, 0)),
            pl.BlockSpec((1, D2), lambda i: (0, 0)),
        ],
        out_specs=pl.BlockSpec((R, D2), lambda i: (i, 0)),
        out_shape=jax.ShapeDtypeStruct((N, D2), _f32),
    )(V0, bias2)


@jax.jit
def _run1(x, edge_index, W1, att_src1, att_dst1):
    loop = jnp.arange(N, dtype=edge_index.dtype)
    pad = jnp.zeros((EP - 320000 - N,), dtype=edge_index.dtype)
    srcP = jnp.concatenate([edge_index[0], loop, pad])
    dstP = jnp.concatenate([edge_index[1], loop, pad])

    TL3, TR3, asd1 = _tc_a(x, W1, att_src1, att_dst1)

    eT = _sc_b(srcP, dstP, asd1)

    zerosC = jnp.zeros((N, W80), _f32)
    TL = TL3.reshape(H1 * N, W80)
    TR = TR3.reshape(H1 * N, W80)
    dummyC = jnp.zeros((CHC, W80), _f32)
    UL, UR = _sc_c(srcP, dstP, eT.reshape(H1 * EP), TL, TR, zerosC, dummyC)
    UL = UL.reshape(H1, N, W80)
    UR = UR.reshape(H1, N, W80)
    return srcP, dstP, zerosC, UL, UR


@jax.jit
def _run2(srcP, dstP, zerosC, UL, UR, bias1, W2, att_src2, att_dst2, bias2):
    h2aug, asd2 = _tc_d(UL, UR, W2, att_src2, att_dst2,
                        bias1.reshape(H1, C1))

    V = _sc_e(srcP, dstP, asd2, h2aug, zerosC)
    Vcat = jnp.concatenate([V[0, :5000], V[1, :5000]], axis=0)

    out2 = _tc_f(Vcat, bias2.reshape(1, D2))
    return out2[:, :D2 // 2], out2[:, D2 // 2:]


def kernel(x, edge_index, W1, att_src1, att_dst1, bias1,
           W2, att_src2, att_dst2, bias2):
    srcP, dstP, zerosC, UL, UR = _run1(x, edge_index, W1, att_src1, att_dst1)
    return _run2(srcP, dstP, zerosC, UL, UR, bias1, W2,
                 att_src2, att_dst2, bias2)


# speedup vs baseline: 20.5476x; 1.0032x over previous
import functools

import jax
import jax.numpy as jnp
from jax import lax
from jax.experimental import pallas as pl
from jax.experimental.pallas import tpu as pltpu
from jax.experimental.pallas import tpu_sc as plsc

N = 10000
IN = 128
H1, C1 = 8, 128
D1A = C1 + 16
D2 = 64
D2A = 80

NC, NS = 2, 16
NW = NC * NS
CH = 688
NCH = 15
EW = CH * NCH
EP = NW * EW
SR = N // NS
NCH_E = 2 * NCH
NE = 5120
SRE = NE // NS
W80 = 80
CHC = 480
NCHC = 20640 // CHC

_i32 = jnp.int32
_f32 = jnp.float32


def _iota16():
    return jax.lax.iota(_i32, 16)


def _tc_a_body(x_ref, w1_ref, as_ref, ad_ref, *out_refs):
    x = x_ref[...]
    s_parts = []
    d_parts = []
    l_parts = []
    r_parts = []
    for h in range(H1):
        w = w1_ref[:, h * C1:(h + 1) * C1]
        ph = jnp.dot(x, w, preferred_element_type=_f32)
        ones = jnp.ones((ph.shape[0], 1), _f32)
        zpad = jnp.zeros((ph.shape[0], 31), _f32)
        l_parts.append(ph[:, :W80][None])
        r_parts.append(jnp.concatenate([ph[:, W80:], ones, zpad], axis=1)[None])
        s_parts.append((ph * as_ref[h][None, :]).sum(axis=1, keepdims=True))
        d_parts.append((ph * ad_ref[h][None, :]).sum(axis=1, keepdims=True))
    out_refs[0][...] = jnp.concatenate(l_parts, axis=0)
    out_refs[1][...] = jnp.concatenate(r_parts, axis=0)
    out_refs[2][...] = jnp.concatenate(s_parts + d_parts, axis=1)


def _tc_a(x, W1, att_src1, att_dst1):
    R = 1000
    grid = (N // R,)
    outs = [jax.ShapeDtypeStruct((H1, N, W80), _f32) for _ in range(2)]
    outs.append(jax.ShapeDtypeStruct((N, 16), _f32))
    out_specs = [pl.BlockSpec((H1, R, W80), lambda i: (0, i, 0))
                 for _ in range(2)]
    out_specs.append(pl.BlockSpec((R, 16), lambda i: (i, 0)))
    return pl.pallas_call(
        _tc_a_body,
        grid=grid,
        in_specs=[
            pl.BlockSpec((R, IN), lambda i: (i, 0)),
            pl.BlockSpec((IN, H1 * C1), lambda i: (0, 0)),
            pl.BlockSpec((H1, C1), lambda i: (0, 0)),
            pl.BlockSpec((H1, C1), lambda i: (0, 0)),
        ],
        out_specs=out_specs,
        out_shape=outs,
    )(x, W1, att_src1, att_dst1)


def _sc_b_body(src_ref, dst_ref, asd_ref, et_ref,
               srcc, dstc, asr, adr, ehv):
    c = lax.axis_index("c")
    s = lax.axis_index("s")
    wid = s * NC + c
    gbase = wid * EW
    iota = _iota16()

    def chunk(k, carry):
        off = gbase + k * CH
        pltpu.sync_copy(src_ref.at[pl.ds(off, CH)], srcc)
        pltpu.sync_copy(dst_ref.at[pl.ds(off, CH)], dstc)
        pltpu.sync_copy(asd_ref.at[srcc], asr)
        pltpu.sync_copy(asd_ref.at[dstc], adr)
        for h in range(H1):
            def body(g, carry2, h=h):
                rows = g * 16 + iota
                a_s = plsc.load_gather(asr, [rows, jnp.full((16,), h, _i32)])
                a_d = plsc.load_gather(adr, [rows, jnp.full((16,), 8 + h, _i32)])
                a = a_s + a_d
                a = jnp.where(a >= 0, a, 0.2 * a)
                e = jnp.exp(a)
                gid = off + rows
                e = jnp.where(gid < (320000 + N), e, jnp.zeros((16,), _f32))
                ehv[pl.ds(g * 16, 16)] = e
                return carry2
            lax.fori_loop(0, CH // 16, body, 0)
            pltpu.sync_copy(ehv, et_ref.at[h, pl.ds(off, CH)])
        return carry

    lax.fori_loop(0, NCH, chunk, 0)


def _sc_b(srcP, dstP, asd1):
    mesh = plsc.VectorSubcoreMesh(core_axis_name="c", subcore_axis_name="s")
    kern = pl.kernel(
        _sc_b_body,
        out_type=jax.ShapeDtypeStruct((H1, EP), _f32),
        mesh=mesh,
        compiler_params=pltpu.CompilerParams(use_tc_tiling_on_sc=False, needs_layout_passes=False),
        scratch_types=[
            pltpu.VMEM((CH,), _i32),
            pltpu.VMEM((CH,), _i32),
            pltpu.VMEM((CH, 16), _f32),
            pltpu.VMEM((CH, 16), _f32),
            pltpu.VMEM((CH,), _f32),
        ],
    )
    return kern(srcP, dstP, asd1)


def _scale_rows(rows, ev, width, count=CH):
    nv = width // 16
    iota = _iota16()

    def body(g, carry):
        evec = ev[pl.ds(g * 16, 16)]
        for lane in range(16):
            e = jnp.full((16,), evec[lane], _f32)
            j = g * 16 + lane
            for cblk in range(nv):
                rows[j, pl.ds(cblk * 16, 16)] = rows[j, pl.ds(cblk * 16, 16)] * e
        return carry

    lax.fori_loop(0, count // 16, body, 0)


def _drain(dummy_hbm, vbuf, sem):
    pltpu.make_async_copy(dummy_hbm, vbuf, sem).wait()


def _sc_c_body(src_ref, dst_ref, et_ref, zeros_ref, dummy_ref, tl_ref, tr_ref,
               u0_ref, u1_ref,
               srcc0, srcc1, dstc0, dstc1, ev0, ev1, rows0, rows1, ush,
               gsem0, gsem1):
    c = lax.axis_index("c")
    s = lax.axis_index("s")
    gbase = s * (NCHC * CHC)
    iota = _iota16()
    srcc = (srcc0, srcc1)
    dstc = (dstc0, dstc1)
    ev = (ev0, ev1)
    rows = (rows0, rows1)
    gsem = (gsem0, gsem1)
    dummy = dummy_ref
    TOT = H1 * NCHC

    def load_src_ev(t, b):
        h = t // NCHC
        off = gbase + (t % NCHC) * CHC
        pltpu.sync_copy(src_ref.at[pl.ds(off, CHC)], srcc[b])
        pltpu.sync_copy(et_ref.at[pl.ds(h * EP + off, CHC)], ev[b])
        hbase = h * N

        def shift(g, carry):
            srcc[b][pl.ds(g * 16, 16)] = srcc[b][pl.ds(g * 16, 16)] + hbase
            return carry

        lax.fori_loop(0, CHC // 16, shift, 0)

    def load_dst(t, b):
        off = gbase + (t % NCHC) * CHC
        pltpu.sync_copy(dst_ref.at[pl.ds(off, CHC)], dstc[b])

    def start_gather(b):
        @pl.when(c == 0)
        def _():
            pltpu.async_copy(tl_ref.at[srcc[b]], rows[b], gsem[b])
        @pl.when(c == 1)
        def _():
            pltpu.async_copy(tr_ref.at[srcc[b]], rows[b], gsem[b])

    pltpu.sync_copy(zeros_ref.at[pl.ds(s * SR, SR)],
                    ush.at[pl.ds(s * SR, SR)])
    plsc.subcore_barrier()
    load_src_ev(0, 0)
    load_dst(0, 0)
    start_gather(0)

    def step(t, carry):
        for b in range(2):
            @pl.when(t % 2 == b)
            def _(b=b):
                nb = 1 - b
                @pl.when(t + 1 < TOT)
                def _():
                    load_src_ev(t + 1, nb)
                    load_dst(t + 1, nb)
                _drain(dummy, rows[b], gsem[b])
                @pl.when(t + 1 < TOT)
                def _():
                    start_gather(nb)
                _scale_rows(rows[b], ev[b], W80, CHC)
                pltpu.sync_copy(rows[b], ush.at[dstc[b]], add=True)
                @pl.when(t % NCHC == NCHC - 1)
                def _():
                    h = t // NCHC
                    plsc.subcore_barrier()
                    @pl.when(c == 0)
                    def _():
                        pltpu.sync_copy(ush.at[pl.ds(s * SR, SR)],
                                        u0_ref.at[pl.ds(h * N + s * SR, SR)])
                    @pl.when(c == 1)
                    def _():
                        pltpu.sync_copy(ush.at[pl.ds(s * SR, SR)],
                                        u1_ref.at[pl.ds(h * N + s * SR, SR)])
                    pltpu.sync_copy(zeros_ref.at[pl.ds(s * SR, SR)],
                                    ush.at[pl.ds(s * SR, SR)])
                    plsc.subcore_barrier()
        return carry

    lax.fori_loop(0, TOT, step, 0)


def _sc_c(srcP, dstP, eT, TL, TR, zerosC, dummyC):
    mesh = plsc.VectorSubcoreMesh(core_axis_name="c", subcore_axis_name="s")
    kern = pl.kernel(
        _sc_c_body,
        out_type=[jax.ShapeDtypeStruct((H1 * N, W80), _f32),
                  jax.ShapeDtypeStruct((H1 * N, W80), _f32)],
        mesh=mesh,
        compiler_params=pltpu.CompilerParams(use_tc_tiling_on_sc=False, needs_layout_passes=False),
        scratch_types=[
            pltpu.VMEM((CHC,), _i32),
            pltpu.VMEM((CHC,), _i32),
            pltpu.VMEM((CHC,), _i32),
            pltpu.VMEM((CHC,), _i32),
            pltpu.VMEM((CHC,), _f32),
            pltpu.VMEM((CHC,), _f32),
            pltpu.VMEM((CHC, W80), _f32),
            pltpu.VMEM((CHC, W80), _f32),
            pltpu.VMEM_SHARED((N, W80), _f32),
            pltpu.SemaphoreType.DMA,
            pltpu.SemaphoreType.DMA,
        ],
    )
    return kern(srcP, dstP, eT, zerosC, dummyC, TL, TR)


def _tc_d_body(ul_ref, ur_ref, w2_ref, as2_ref, ad2_ref, b1_ref,
               h2_ref, asd2_ref):
    parts = []
    for h in range(H1):
        ul = ul_ref[h]
        ur = ur_ref[h]
        den = ur[:, 48:49] + 1e-16
        xh = jnp.concatenate([ul, ur[:, :48]], axis=1) / den + b1_ref[h][None, :]
        parts.append(jnp.maximum(xh, 0.0))
    x1 = jnp.concatenate(parts, axis=1)
    h2 = jnp.dot(x1, w2_ref[...], preferred_element_type=_f32)
    s2 = (h2 * as2_ref[...]).sum(axis=1, keepdims=True)
    d2 = (h2 * ad2_ref[...]).sum(axis=1, keepdims=True)
    ones = jnp.ones((h2.shape[0], 1), _f32)
    zp = jnp.zeros((h2.shape[0], 15), _f32)
    h2_ref[...] = jnp.concatenate([h2, ones, zp], axis=1)
    asd2_ref[...] = jnp.concatenate([s2, d2, jnp.zeros((h2.shape[0], 14), _f32)],
                                    axis=1)


def _tc_d(UL, UR, W2, att_src2, att_dst2, bias1):
    R = 1000
    grid = (N // R,)
    return pl.pallas_call(
        _tc_d_body,
        grid=grid,
        in_specs=[
            pl.BlockSpec((H1, R, W80), lambda i: (0, i, 0)),
            pl.BlockSpec((H1, R, W80), lambda i: (0, i, 0)),
            pl.BlockSpec((H1 * C1, D2), lambda i: (0, 0)),
            pl.BlockSpec((1, D2), lambda i: (0, 0)),
            pl.BlockSpec((1, D2), lambda i: (0, 0)),
            pl.BlockSpec((H1, C1), lambda i: (0, 0)),
        ],
        out_specs=[
            pl.BlockSpec((R, D2A), lambda i: (i, 0)),
            pl.BlockSpec((R, 16), lambda i: (i, 0)),
        ],
        out_shape=[
            jax.ShapeDtypeStruct((N, D2A), _f32),
            jax.ShapeDtypeStruct((N, 16), _f32),
        ],
    )(UL, UR, W2, att_src2, att_dst2, bias1)


def _sc_e_body(src_ref, dst_ref, asd_ref, h2_ref, zeros_ref, u_ref,
               srcc, dstc, asr, adr, ev, rows, ush):
    c = lax.axis_index("c")
    s = lax.axis_index("s")
    gbase = s * (NCH_E * CH)
    iota = _iota16()
    pltpu.sync_copy(zeros_ref.at[pl.ds(s * SRE, SRE)],
                    ush.at[pl.ds(s * SRE, SRE)])
    plsc.subcore_barrier()
    def chunk(k, carry):
        off = gbase + k * CH
        pltpu.sync_copy(src_ref.at[pl.ds(off, CH)], srcc)
        pltpu.sync_copy(dst_ref.at[pl.ds(off, CH)], dstc)
        pltpu.sync_copy(asd_ref.at[srcc], asr)
        pltpu.sync_copy(asd_ref.at[dstc], adr)

        def body(g, carry2):
            rws = g * 16 + iota
            a_s = plsc.load_gather(asr, [rws, jnp.zeros((16,), _i32)])
            a_d = plsc.load_gather(adr, [rws, jnp.ones((16,), _i32)])
            a = a_s + a_d
            a = jnp.where(a >= 0, a, 0.2 * a)
            e = jnp.exp(a)
            gid = off + rws
            e = jnp.where(gid < (320000 + N), e, jnp.zeros((16,), _f32))
            ev[pl.ds(g * 16, 16)] = e
            d = dstc[pl.ds(g * 16, 16)] - c * 5000
            d = jnp.where((d >= 0) & (d < 5000), d,
                          jnp.full((16,), 5000, _i32))
            dstc[pl.ds(g * 16, 16)] = d
            return carry2

        lax.fori_loop(0, CH // 16, body, 0)
        pltpu.sync_copy(h2_ref.at[srcc], rows)
        _scale_rows(rows, ev, D2A)
        pltpu.sync_copy(rows, ush.at[dstc], add=True)
        return carry

    lax.fori_loop(0, NCH_E, chunk, 0)
    plsc.subcore_barrier()
    pltpu.sync_copy(ush.at[pl.ds(s * SRE, SRE)],
                    u_ref.at[c, pl.ds(s * SRE, SRE)])


def _sc_e(srcP, dstP, asd2, h2aug, zerosB):
    mesh = plsc.VectorSubcoreMesh(core_axis_name="c", subcore_axis_name="s")
    kern = pl.kernel(
        _sc_e_body,
        out_type=jax.ShapeDtypeStruct((NC, NE, D2A), _f32),
        mesh=mesh,
        compiler_params=pltpu.CompilerParams(use_tc_tiling_on_sc=False, needs_layout_passes=False),
        scratch_types=[
            pltpu.VMEM((CH,), _i32),
            pltpu.VMEM((CH,), _i32),
            pltpu.VMEM((CH, 16), _f32),
            pltpu.VMEM((CH, 16), _f32),
            pltpu.VMEM((CH,), _f32),
            pltpu.VMEM((CH, D2A), _f32),
            pltpu.VMEM_SHARED((NE, D2A), _f32),
        ],
    )
    return kern(srcP, dstP, asd2, h2aug, zerosB)


def _tc_f_body(u0_ref, b2_ref, out_ref):
    u = u0_ref[...]
    den = u[:, D2:D2 + 1] + 1e-16
    out_ref[...] = u[:, :D2] / den + b2_ref[...]


def _tc_f(V0, bias2):
    R = 1000
    return pl.pallas_call(
        _tc_f_body,
        grid=(N // R,),
        in_specs=[
            pl.BlockSpec((R, D2A), lambda i: (i, 0)),
            pl.BlockSpec((1, D2), lambda i: (0, 0)),
        ],
        out_specs=pl.BlockSpec((R, D2), lambda i: (i, 0)),
        out_shape=jax.ShapeDtypeStruct((N, D2), _f32),
    )(V0, bias2)


@jax.jit
def _run1(x, edge_index, W1, att_src1, att_dst1):
    loop = jnp.arange(N, dtype=edge_index.dtype)
    pad = jnp.zeros((EP - 320000 - N,), dtype=edge_index.dtype)
    srcP = jnp.concatenate([edge_index[0], loop, pad])
    dstP = jnp.concatenate([edge_index[1], loop, pad])

    TL3, TR3, asd1 = _tc_a(x, W1, att_src1, att_dst1)

    eT = _sc_b(srcP, dstP, asd1)

    zerosC = jnp.zeros((N, W80), _f32)
    TL = TL3.reshape(H1 * N, W80)
    TR = TR3.reshape(H1 * N, W80)
    dummyC = jnp.zeros((CHC, W80), _f32)
    UL, UR = _sc_c(srcP, dstP, eT.reshape(H1 * EP), TL, TR, zerosC, dummyC)
    UL = UL.reshape(H1, N, W80)
    UR = UR.reshape(H1, N, W80)
    return srcP, dstP, zerosC, UL, UR


@jax.jit
def _run2(srcP, dstP, zerosC, UL, UR, bias1, W2, att_src2, att_dst2, bias2):
    h2aug, asd2 = _tc_d(UL, UR, W2, att_src2, att_dst2,
                        bias1.reshape(H1, C1))

    V = _sc_e(srcP, dstP, asd2, h2aug, zerosC)
    Vcat = jnp.concatenate([V[0, :5000], V[1, :5000]], axis=0)

    out2 = _tc_f(Vcat, bias2.reshape(1, D2))
    return out2[:, :D2 // 2], out2[:, D2 // 2:]


def kernel(x, edge_index, W1, att_src1, att_dst1, bias1,
           W2, att_src2, att_dst2, bias2):
    srcP, dstP, zerosC, UL, UR = _run1(x, edge_index, W1, att_src1, att_dst1)
    return _run2(srcP, dstP, zerosC, UL, UR, bias1, W2,
                 att_src2, att_dst2, bias2)
